# Initial kernel scaffold; baseline (speedup 1.0000x reference)
import jax
import jax.numpy as jnp
from jax.experimental import pallas as pl


def kernel(x, W_embed, b_embed, W_diag, W_chr, b_chr, W_ecc_proj, b_ecc_proj, W_e1, b_e1, W_e2, b_e2):
    raise NotImplementedError("write your pallas kernel here")


# speedup vs baseline: 4.7031x; 4.7031x over previous
import functools

import jax
import jax.numpy as jnp
from jax.experimental import pallas as pl
from jax.experimental.pallas import tpu as pltpu

D_MODEL = 1024
D_EMBED = 128
N_CHR = 32
ECC_BITS = 32
K = 32
BATCH = 4
SEQ = 2048

ROWS_PER_BLOCK = 512
SIM_ROWS = 256


def _proj_body(x_ref, we_ref, be_ref, wd_ref, wc_ref, bc_ref, wp_ref, bp_ref,
               w1_ref, b1_ref, w2_ref, b2_ref,
               emb_ref, embn_ref, met_ref, chr_ref, ecc_ref):
    x = x_ref[...]
    emb = jnp.dot(x, we_ref[...], preferred_element_type=jnp.float32) + be_ref[...]
    emb_ref[...] = emb
    nrm = jnp.sqrt(jnp.sum(emb * emb, axis=-1, keepdims=True)) + 1e-8
    embn_ref[...] = emb / nrm
    met_ref[...] = jnp.dot(x, wd_ref[...], preferred_element_type=jnp.float32)
    chr_ref[...] = jnp.dot(x, wc_ref[...], preferred_element_type=jnp.float32) + bc_ref[...]
    p = jnp.dot(x, wp_ref[...], preferred_element_type=jnp.float32) + bp_ref[...]
    h = jnp.tanh(jnp.dot(p, w1_ref[...], preferred_element_type=jnp.float32) + b1_ref[...])
    e = jnp.dot(h, w2_ref[...], preferred_element_type=jnp.float32) + b2_ref[...]
    ecc_ref[...] = 1.0 / (1.0 + jnp.exp(-e))


def _simtopk_body(rows_ref, cols_ref, sc_ref, ix_ref):
    i = pl.program_id(1)
    rows = rows_ref[0]
    cols = cols_ref[0]
    sim = jax.lax.dot_general(rows, cols, (((1,), (1,)), ((), ())),
                              preferred_element_type=jnp.float32)
    col_iota = jax.lax.broadcasted_iota(jnp.int32, (SIM_ROWS, SEQ), 1)
    row_ids = i * SIM_ROWS + jax.lax.broadcasted_iota(jnp.int32, (SIM_ROWS, SEQ), 0)
    sim = jnp.where(col_iota == row_ids, sim - 1e9, sim)
    scores = []
    idxs = []
    neg = jnp.float32(-jnp.inf)
    for _ in range(K):
        m = jnp.max(sim, axis=1, keepdims=True)
        am = jnp.min(jnp.where(sim == m, col_iota, SEQ), axis=1, keepdims=True)
        scores.append(m)
        idxs.append(am)
        sim = jnp.where(col_iota == am, neg, sim)
    sc_ref[0] = jnp.concatenate(scores, axis=1)
    ix_ref[0] = jnp.concatenate(idxs, axis=1)


def kernel(x, W_embed, b_embed, W_diag, W_chr, b_chr,
           W_ecc_proj, b_ecc_proj, W_e1, b_e1, W_e2, b_e2):
    xf = x.reshape(BATCH * SEQ, D_MODEL)
    nblk = (BATCH * SEQ) // ROWS_PER_BLOCK

    def _full(shape):
        return pl.BlockSpec(shape, lambda i: (0,) * len(shape))

    emb, embn, met, chrs, ecc = pl.pallas_call(
        _proj_body,
        grid=(nblk,),
        in_specs=[
            pl.BlockSpec((ROWS_PER_BLOCK, D_MODEL), lambda i: (i, 0)),
            _full((D_MODEL, D_EMBED)), _full((1, D_EMBED)),
            _full((D_MODEL, D_MODEL)),
            _full((D_MODEL, N_CHR)), _full((1, N_CHR)),
            _full((D_MODEL, ECC_BITS)), _full((1, ECC_BITS)),
            _full((ECC_BITS, ECC_BITS * 2)), _full((1, ECC_BITS * 2)),
            _full((ECC_BITS * 2, ECC_BITS)), _full((1, ECC_BITS)),
        ],
        out_specs=[
            pl.BlockSpec((ROWS_PER_BLOCK, D_EMBED), lambda i: (i, 0)),
            pl.BlockSpec((ROWS_PER_BLOCK, D_EMBED), lambda i: (i, 0)),
            pl.BlockSpec((ROWS_PER_BLOCK, D_MODEL), lambda i: (i, 0)),
            pl.BlockSpec((ROWS_PER_BLOCK, N_CHR), lambda i: (i, 0)),
            pl.BlockSpec((ROWS_PER_BLOCK, ECC_BITS), lambda i: (i, 0)),
        ],
        out_shape=[
            jax.ShapeDtypeStruct((BATCH * SEQ, D_EMBED), jnp.float32),
            jax.ShapeDtypeStruct((BATCH * SEQ, D_EMBED), jnp.float32),
            jax.ShapeDtypeStruct((BATCH * SEQ, D_MODEL), jnp.float32),
            jax.ShapeDtypeStruct((BATCH * SEQ, N_CHR), jnp.float32),
            jax.ShapeDtypeStruct((BATCH * SEQ, ECC_BITS), jnp.float32),
        ],
    )(xf, W_embed, b_embed.reshape(1, -1), W_diag, W_chr, b_chr.reshape(1, -1),
      W_ecc_proj, b_ecc_proj.reshape(1, -1), W_e1, b_e1.reshape(1, -1),
      W_e2, b_e2.reshape(1, -1))

    embn3 = embn.reshape(BATCH, SEQ, D_EMBED)
    nrow = SEQ // SIM_ROWS
    knn_scores, knn_indices = pl.pallas_call(
        _simtopk_body,
        grid=(BATCH, nrow),
        in_specs=[
            pl.BlockSpec((1, SIM_ROWS, D_EMBED), lambda b, i: (b, i, 0)),
            pl.BlockSpec((1, SEQ, D_EMBED), lambda b, i: (b, 0, 0)),
        ],
        out_specs=[
            pl.BlockSpec((1, SIM_ROWS, K), lambda b, i: (b, i, 0)),
            pl.BlockSpec((1, SIM_ROWS, K), lambda b, i: (b, i, 0)),
        ],
        out_shape=[
            jax.ShapeDtypeStruct((BATCH, SEQ, K), jnp.float32),
            jax.ShapeDtypeStruct((BATCH, SEQ, K), jnp.int32),
        ],
    )(embn3, embn3)

    embedding = emb.reshape(BATCH, SEQ, D_EMBED)
    metric = met.reshape(BATCH, SEQ, D_MODEL)
    christoffel = chrs.reshape(BATCH, SEQ, N_CHR)
    ecc_out = ecc.reshape(BATCH, SEQ, ECC_BITS)
    half_k = K // 2
    min_heap = knn_scores[..., :half_k]
    max_heap = -knn_scores[..., half_k:]
    return (embedding, metric, christoffel, knn_scores, knn_indices,
            min_heap, max_heap, ecc_out)
